# baseline (speedup 1.0000x reference)
import jax
import jax.numpy as jnp
from jax import lax
from jax.experimental import pallas as pl
from jax.experimental.pallas import tpu as pltpu
from jax.experimental.pallas import tpu_sc as plsc

B, C, H, W = 2, 384, 224, 224
OH, OW, K = 112, 112, 4
BC = B * C
OP = OH * OW

NC, NS = 2, 16
NW = NC * NS
PLANES_PER_W = BC // NW
NPPAIR = PLANES_PER_W // 2
ROWS = 8
NCHUNK = OH // ROWS
CHUNK = ROWS * OW
SMROW = 2 * K * CHUNK
NCPAIR = NCHUNK // 2


def _body(x_hbm, sm_hbm, out_hbm,
          pba, pbb, sma, smb, oba, obb,
          psem_a, psem_b, ssem_a, ssem_b, osem_a, osem_b):
    wid = lax.axis_index("s") * NC + lax.axis_index("c")
    plane_base = wid * PLANES_PER_W

    def plane_start(p, pbuf, sem):
        b = jnp.where(p >= C, 1, 0)
        c = p - b * C
        pltpu.async_copy(x_hbm.at[b, c], pbuf, sem)

    def plane_wait(pbuf, sem):
        pltpu.make_async_copy(x_hbm.at[0, 0], pbuf, sem).wait()

    def sm_start(ch, buf, sem):
        pltpu.async_copy(sm_hbm.at[pl.ds(ch * SMROW, SMROW)], buf, sem)

    def sm_wait(buf, sem):
        pltpu.make_async_copy(sm_hbm.at[pl.ds(0, SMROW)], buf, sem).wait()

    def out_start(bb, cc, ch, obuf, sem):
        pltpu.async_copy(obuf, out_hbm.at[bb, cc, pl.ds(ch * ROWS, ROWS), :], sem)

    def out_drain(obuf, sem):
        pltpu.make_async_copy(out_hbm.at[0, 0, pl.ds(0, ROWS), :], obuf, sem).wait()

    def compute_chunk(pbuf, smbuf, obuf):
        @plsc.parallel_loop(0, CHUNK, step=16, unroll=2)
        def _(base):
            q = jnp.right_shift(base, 4)
            row = jnp.right_shift(q * 9363, 16)
            col = base - row * OW
            acc = jnp.zeros((16,), jnp.float32)
            for k in range(K):
                xs = smbuf[pl.ds((2 * k) * CHUNK + base, 16)]
                ys = smbuf[pl.ds((2 * k + 1) * CHUNK + base, 16)]
                x0 = jnp.clip(xs.astype(jnp.int32), 0, W - 2)
                y0 = jnp.clip(ys.astype(jnp.int32), 0, H - 2)
                wx = xs - x0.astype(jnp.float32)
                wy = ys - y0.astype(jnp.float32)
                x1 = x0 + 1
                y1 = y0 + 1
                v00 = plsc.load_gather(pbuf, [y0, x0])
                v01 = plsc.load_gather(pbuf, [y0, x1])
                v10 = plsc.load_gather(pbuf, [y1, x0])
                v11 = plsc.load_gather(pbuf, [y1, x1])
                t0 = v00 + wx * (v01 - v00)
                t1 = v10 + wx * (v11 - v10)
                acc = acc + (t0 + wy * (t1 - t0))
            obuf[row, pl.ds(col, 16)] = acc * 0.25

    def process_plane(p, pbuf, first):
        bb = jnp.where(p >= C, 1, 0)
        cc = p - bb * C
        sm_start(0, sma, ssem_a)

        def cpair_body(ci, _):
            c0 = ci * 2
            sm_start(c0 + 1, smb, ssem_b)
            sm_wait(sma, ssem_a)

            @pl.when(jnp.logical_or(jnp.logical_not(first), ci > 0))
            def _():
                out_drain(oba, osem_a)

            compute_chunk(pbuf, sma, oba)
            out_start(bb, cc, c0, oba, osem_a)

            @pl.when(ci < NCPAIR - 1)
            def _():
                sm_start(c0 + 2, sma, ssem_a)

            sm_wait(smb, ssem_b)

            @pl.when(jnp.logical_or(jnp.logical_not(first), ci > 0))
            def _():
                out_drain(obb, osem_b)

            compute_chunk(pbuf, smb, obb)
            out_start(bb, cc, c0 + 1, obb, osem_b)
            return 0

        lax.fori_loop(0, NCPAIR, cpair_body, 0)

    plane_start(plane_base, pba, psem_a)
    plane_start(plane_base + 1, pbb, psem_b)

    def ppair_body(i, _):
        p0 = plane_base + i * 2
        plane_wait(pba, psem_a)
        process_plane(p0, pba, i == 0)

        @pl.when(i < NPPAIR - 1)
        def _():
            plane_start(p0 + 2, pba, psem_a)

        plane_wait(pbb, psem_b)
        process_plane(p0 + 1, pbb, jnp.bool_(False))

        @pl.when(i < NPPAIR - 1)
        def _():
            plane_start(p0 + 3, pbb, psem_b)

        return 0

    lax.fori_loop(0, NPPAIR, ppair_body, 0)
    out_drain(oba, osem_a)
    out_drain(obb, osem_b)


@jax.jit
def _mapped_avg_pool_sc(x, sm_t):
    k = pl.kernel(
        _body,
        out_type=jax.ShapeDtypeStruct((B, C, OH, OW), jnp.float32),
        mesh=plsc.VectorSubcoreMesh(core_axis_name="c", subcore_axis_name="s"),
        scratch_types=[
            pltpu.VMEM((H, W), jnp.float32),
            pltpu.VMEM((H, W), jnp.float32),
            pltpu.VMEM((SMROW,), jnp.float32),
            pltpu.VMEM((SMROW,), jnp.float32),
            pltpu.VMEM((ROWS, OW), jnp.float32),
            pltpu.VMEM((ROWS, OW), jnp.float32),
            pltpu.SemaphoreType.DMA,
            pltpu.SemaphoreType.DMA,
            pltpu.SemaphoreType.DMA,
            pltpu.SemaphoreType.DMA,
            pltpu.SemaphoreType.DMA,
            pltpu.SemaphoreType.DMA,
        ],
        compiler_params=pltpu.CompilerParams(needs_layout_passes=False),
    )
    return k(x, sm_t)


def kernel(x, sample_map):
    smf = sample_map.reshape(NCHUNK, CHUNK, 2 * K)
    sm_t = smf.transpose(0, 2, 1).reshape(NCHUNK * SMROW)
    return _mapped_avg_pool_sc(x, sm_t)


# speedup vs baseline: 1.1469x; 1.1469x over previous
import jax
import jax.numpy as jnp
from jax import lax
from jax.experimental import pallas as pl
from jax.experimental.pallas import tpu as pltpu
from jax.experimental.pallas import tpu_sc as plsc

B, C, H, W = 2, 384, 224, 224
OH, OW, K = 112, 112, 4
BC = B * C
OP = OH * OW

NC, NS = 2, 16
NW = NC * NS
PLANES_PER_W = BC // NW
NPPAIR = PLANES_PER_W // 2
ROWS = 8
NCHUNK = OH // ROWS
CHUNK = ROWS * OW
HPIX = CHUNK // 2
SMH = 2 * K * HPIX
NCPAIR = NCHUNK // 2


def _body(x_hbm, sm_hbm, out_hbm,
          pba, pbb, sma, smb, oa0, oa1, ob0, ob1,
          psem, ssem_a, ssem_b, osem_a0, osem_a1, osem_b0, osem_b1):
    wid = lax.axis_index("s") * NC + lax.axis_index("c")
    plane_base = wid * PLANES_PER_W

    def bc_of(p):
        b = jnp.where(p >= C, 1, 0)
        return b, p - b * C

    def sm_start(h, buf, sem):
        pltpu.async_copy(sm_hbm.at[pl.ds(h * SMH, SMH)], buf, sem)

    def sm_wait(buf, sem):
        pltpu.make_async_copy(sm_hbm.at[pl.ds(0, SMH)], buf, sem).wait()

    def out_start(bb, cc, ch, obuf, sem):
        pltpu.async_copy(obuf, out_hbm.at[bb, cc, pl.ds(ch * ROWS, ROWS), :], sem)

    def out_drain(obuf, sem):
        pltpu.make_async_copy(out_hbm.at[0, 0, pl.ds(0, ROWS), :], obuf, sem).wait()

    def compute_half(smbuf, half, obufa, obufb):
        for k in range(K):
            @plsc.parallel_loop(0, HPIX, step=16, unroll=4)
            def _(base):
                q = jnp.right_shift(base, 4)
                rl = jnp.right_shift(q * 9363, 16)
                row = rl + half * 4
                col = base - rl * OW
                xs = smbuf[pl.ds((2 * k) * HPIX + base, 16)]
                ys = smbuf[pl.ds((2 * k + 1) * HPIX + base, 16)]
                x0 = jnp.clip(xs.astype(jnp.int32), 0, W - 2)
                y0 = jnp.clip(ys.astype(jnp.int32), 0, H - 2)
                wx = xs - x0.astype(jnp.float32)
                wy = ys - y0.astype(jnp.float32)
                x1 = x0 + 1
                y1 = y0 + 1
                for pbuf, obuf in ((pba, obufa), (pbb, obufb)):
                    v00 = plsc.load_gather(pbuf, [y0, x0])
                    v01 = plsc.load_gather(pbuf, [y0, x1])
                    v10 = plsc.load_gather(pbuf, [y1, x0])
                    v11 = plsc.load_gather(pbuf, [y1, x1])
                    t0 = v00 + wx * (v01 - v00)
                    t1 = v10 + wx * (v11 - v10)
                    val = (t0 + wy * (t1 - t0)) * 0.25
                    if k == 0:
                        obuf[row, pl.ds(col, 16)] = val
                    else:
                        plsc.addupdate(obuf.at[row, pl.ds(col, 16)], val)

    def ppair_body(i, _):
        p0 = plane_base + i * 2
        ba, ca = bc_of(p0)
        bb, cb = bc_of(p0 + 1)
        pltpu.async_copy(x_hbm.at[ba, ca], pba, psem)
        pltpu.async_copy(x_hbm.at[bb, cb], pbb, psem)
        sm_start(0, sma, ssem_a)
        pltpu.make_async_copy(x_hbm.at[0, 0], pba, psem).wait()
        pltpu.make_async_copy(x_hbm.at[0, 0], pbb, psem).wait()

        def cpair_body(ci, _):
            h0 = ci * 4
            first = jnp.logical_and(i == 0, ci == 0)

            sm_start(h0 + 1, smb, ssem_b)
            sm_wait(sma, ssem_a)

            @pl.when(jnp.logical_not(first))
            def _():
                out_drain(oa0, osem_a0)
                out_drain(ob0, osem_b0)

            compute_half(sma, 0, oa0, ob0)
            sm_start(h0 + 2, sma, ssem_a)
            sm_wait(smb, ssem_b)
            compute_half(smb, 1, oa0, ob0)
            out_start(ba, ca, ci * 2, oa0, osem_a0)
            out_start(bb, cb, ci * 2, ob0, osem_b0)

            sm_start(h0 + 3, smb, ssem_b)
            sm_wait(sma, ssem_a)

            @pl.when(jnp.logical_not(first))
            def _():
                out_drain(oa1, osem_a1)
                out_drain(ob1, osem_b1)

            compute_half(sma, 0, oa1, ob1)

            @pl.when(ci < NCPAIR - 1)
            def _():
                sm_start(h0 + 4, sma, ssem_a)

            sm_wait(smb, ssem_b)
            compute_half(smb, 1, oa1, ob1)
            out_start(ba, ca, ci * 2 + 1, oa1, osem_a1)
            out_start(bb, cb, ci * 2 + 1, ob1, osem_b1)
            return 0

        lax.fori_loop(0, NCPAIR, cpair_body, 0)
        return 0

    lax.fori_loop(0, NPPAIR, ppair_body, 0)
    out_drain(oa0, osem_a0)
    out_drain(ob0, osem_b0)
    out_drain(oa1, osem_a1)
    out_drain(ob1, osem_b1)


@jax.jit
def _mapped_avg_pool_sc(x, sm_t):
    k = pl.kernel(
        _body,
        out_type=jax.ShapeDtypeStruct((B, C, OH, OW), jnp.float32),
        mesh=plsc.VectorSubcoreMesh(core_axis_name="c", subcore_axis_name="s"),
        scratch_types=[
            pltpu.VMEM((H, W), jnp.float32),
            pltpu.VMEM((H, W), jnp.float32),
            pltpu.VMEM((SMH,), jnp.float32),
            pltpu.VMEM((SMH,), jnp.float32),
            pltpu.VMEM((ROWS, OW), jnp.float32),
            pltpu.VMEM((ROWS, OW), jnp.float32),
            pltpu.VMEM((ROWS, OW), jnp.float32),
            pltpu.VMEM((ROWS, OW), jnp.float32),
            pltpu.SemaphoreType.DMA,
            pltpu.SemaphoreType.DMA,
            pltpu.SemaphoreType.DMA,
            pltpu.SemaphoreType.DMA,
            pltpu.SemaphoreType.DMA,
            pltpu.SemaphoreType.DMA,
            pltpu.SemaphoreType.DMA,
        ],
        compiler_params=pltpu.CompilerParams(needs_layout_passes=False),
    )
    return k(x, sm_t)


def kernel(x, sample_map):
    smf = sample_map.reshape(NCHUNK * 2, HPIX, 2 * K)
    sm_t = smf.transpose(0, 2, 1).reshape(NCHUNK * 2 * SMH)
    return _mapped_avg_pool_sc(x, sm_t)


# speedup vs baseline: 1.3158x; 1.1473x over previous
import jax
import jax.numpy as jnp
from jax import lax
from jax.experimental import pallas as pl
from jax.experimental.pallas import tpu as pltpu
from jax.experimental.pallas import tpu_sc as plsc

B, C, H, W = 2, 384, 224, 224
OH, OW, K = 112, 112, 4
BC = B * C
OP = OH * OW

NC, NS = 2, 16
NW = NC * NS
PLANES_PER_W = BC // NW
NPPAIR = PLANES_PER_W // 2
ROWS = 8
NCHUNK = OH // ROWS
CHUNK = ROWS * OW
HPIX = CHUNK // 2
SMH = 2 * K * HPIX
NCPAIR = NCHUNK // 2


def _body(x_hbm, sm_hbm, out_hbm,
          pba, pbb, sma, smb, oa0, oa1, ob0, ob1,
          psem, ssem_a, ssem_b, osem_a0, osem_a1, osem_b0, osem_b1):
    wid = lax.axis_index("s") * NC + lax.axis_index("c")
    plane_base = wid * PLANES_PER_W

    def bc_of(p):
        b = jnp.where(p >= C, 1, 0)
        return b, p - b * C

    def sm_start(h, buf, sem):
        pltpu.async_copy(sm_hbm.at[pl.ds(h * SMH, SMH)], buf, sem)

    def sm_wait(buf, sem):
        pltpu.make_async_copy(sm_hbm.at[pl.ds(0, SMH)], buf, sem).wait()

    def out_start(bb, cc, ch, obuf, sem):
        pltpu.async_copy(obuf, out_hbm.at[bb, cc, pl.ds(ch * ROWS, ROWS), :], sem)

    def out_drain(obuf, sem):
        pltpu.make_async_copy(out_hbm.at[0, 0, pl.ds(0, ROWS), :], obuf, sem).wait()

    def compute_half(smbuf, half, obufa, obufb):
        @plsc.parallel_loop(0, HPIX, step=16, unroll=2)
        def _(base):
            q = jnp.right_shift(base, 4)
            rl = jnp.right_shift(q * 9363, 16)
            row = rl + half * 4
            col = base - rl * OW
            acc0 = jnp.zeros((16,), jnp.float32)
            acc1 = jnp.zeros((16,), jnp.float32)
            for k in range(K):
                xs = smbuf[pl.ds((2 * k) * HPIX + base, 16)]
                ys = smbuf[pl.ds((2 * k + 1) * HPIX + base, 16)]
                x0 = jnp.clip(xs.astype(jnp.int32), 0, W - 2)
                y0 = jnp.clip(ys.astype(jnp.int32), 0, H - 2)
                wx = xs - x0.astype(jnp.float32)
                wy = ys - y0.astype(jnp.float32)
                x1 = x0 + 1
                y1 = y0 + 1
                for pbuf, which in ((pba, 0), (pbb, 1)):
                    v00 = plsc.load_gather(pbuf, [y0, x0])
                    v01 = plsc.load_gather(pbuf, [y0, x1])
                    v10 = plsc.load_gather(pbuf, [y1, x0])
                    v11 = plsc.load_gather(pbuf, [y1, x1])
                    t0 = v00 + wx * (v01 - v00)
                    t1 = v10 + wx * (v11 - v10)
                    val = t0 + wy * (t1 - t0)
                    if which == 0:
                        acc0 = acc0 + val
                    else:
                        acc1 = acc1 + val
            obufa[row, pl.ds(col, 16)] = acc0 * 0.25
            obufb[row, pl.ds(col, 16)] = acc1 * 0.25

    def ppair_body(i, _):
        p0 = plane_base + i * 2
        ba, ca = bc_of(p0)
        bb, cb = bc_of(p0 + 1)
        pltpu.async_copy(x_hbm.at[ba, ca], pba, psem)
        pltpu.async_copy(x_hbm.at[bb, cb], pbb, psem)
        sm_start(0, sma, ssem_a)
        pltpu.make_async_copy(x_hbm.at[0, 0], pba, psem).wait()
        pltpu.make_async_copy(x_hbm.at[0, 0], pbb, psem).wait()

        def cpair_body(ci, _):
            h0 = ci * 4
            first = jnp.logical_and(i == 0, ci == 0)

            sm_start(h0 + 1, smb, ssem_b)
            sm_wait(sma, ssem_a)

            @pl.when(jnp.logical_not(first))
            def _():
                out_drain(oa0, osem_a0)
                out_drain(ob0, osem_b0)

            compute_half(sma, 0, oa0, ob0)
            sm_start(h0 + 2, sma, ssem_a)
            sm_wait(smb, ssem_b)
            compute_half(smb, 1, oa0, ob0)
            out_start(ba, ca, ci * 2, oa0, osem_a0)
            out_start(bb, cb, ci * 2, ob0, osem_b0)

            sm_start(h0 + 3, smb, ssem_b)
            sm_wait(sma, ssem_a)

            @pl.when(jnp.logical_not(first))
            def _():
                out_drain(oa1, osem_a1)
                out_drain(ob1, osem_b1)

            compute_half(sma, 0, oa1, ob1)

            @pl.when(ci < NCPAIR - 1)
            def _():
                sm_start(h0 + 4, sma, ssem_a)

            sm_wait(smb, ssem_b)
            compute_half(smb, 1, oa1, ob1)
            out_start(ba, ca, ci * 2 + 1, oa1, osem_a1)
            out_start(bb, cb, ci * 2 + 1, ob1, osem_b1)
            return 0

        lax.fori_loop(0, NCPAIR, cpair_body, 0)
        return 0

    lax.fori_loop(0, NPPAIR, ppair_body, 0)
    out_drain(oa0, osem_a0)
    out_drain(ob0, osem_b0)
    out_drain(oa1, osem_a1)
    out_drain(ob1, osem_b1)


@jax.jit
def _mapped_avg_pool_sc(x, sm_t):
    k = pl.kernel(
        _body,
        out_type=jax.ShapeDtypeStruct((B, C, OH, OW), jnp.float32),
        mesh=plsc.VectorSubcoreMesh(core_axis_name="c", subcore_axis_name="s"),
        scratch_types=[
            pltpu.VMEM((H, W), jnp.float32),
            pltpu.VMEM((H, W), jnp.float32),
            pltpu.VMEM((SMH,), jnp.float32),
            pltpu.VMEM((SMH,), jnp.float32),
            pltpu.VMEM((ROWS, OW), jnp.float32),
            pltpu.VMEM((ROWS, OW), jnp.float32),
            pltpu.VMEM((ROWS, OW), jnp.float32),
            pltpu.VMEM((ROWS, OW), jnp.float32),
            pltpu.SemaphoreType.DMA,
            pltpu.SemaphoreType.DMA,
            pltpu.SemaphoreType.DMA,
            pltpu.SemaphoreType.DMA,
            pltpu.SemaphoreType.DMA,
            pltpu.SemaphoreType.DMA,
            pltpu.SemaphoreType.DMA,
        ],
        compiler_params=pltpu.CompilerParams(needs_layout_passes=False),
    )
    return k(x, sm_t)


def kernel(x, sample_map):
    smf = sample_map.reshape(NCHUNK * 2, HPIX, 2 * K)
    sm_t = smf.transpose(0, 2, 1).reshape(NCHUNK * 2 * SMH)
    return _mapped_avg_pool_sc(x, sm_t)


# speedup vs baseline: 1.3309x; 1.0115x over previous
import jax
import jax.numpy as jnp
from jax import lax
from jax.experimental import pallas as pl
from jax.experimental.pallas import tpu as pltpu
from jax.experimental.pallas import tpu_sc as plsc

B, C, H, W = 2, 384, 224, 224
OH, OW, K = 112, 112, 4
BC = B * C
OP = OH * OW

NC, NS = 2, 16
NW = NC * NS
PLANES_PER_W = BC // NW
NPPAIR = PLANES_PER_W // 2
ROWS = 8
NCHUNK = OH // ROWS
CHUNK = ROWS * OW
HPIX = CHUNK // 2
SMH = 2 * K * HPIX
NCPAIR = NCHUNK // 2


def _body(x_hbm, sm_hbm, out_hbm,
          pba, pbb, sma, smb, oa0, oa1, ob0, ob1,
          psem, ssem_a, ssem_b, osem_a0, osem_a1, osem_b0, osem_b1):
    wid = lax.axis_index("s") * NC + lax.axis_index("c")
    plane_base = wid * PLANES_PER_W

    def bc_of(p):
        b = jnp.where(p >= C, 1, 0)
        return b, p - b * C

    def sm_start(h, buf, sem):
        pltpu.async_copy(sm_hbm.at[pl.ds(h * SMH, SMH)], buf, sem)

    def sm_wait(buf, sem):
        pltpu.make_async_copy(sm_hbm.at[pl.ds(0, SMH)], buf, sem).wait()

    def out_start(bb, cc, ch, obuf, sem):
        pltpu.async_copy(obuf, out_hbm.at[bb, cc, pl.ds(ch * ROWS, ROWS), :], sem)

    def out_drain(obuf, sem):
        pltpu.make_async_copy(out_hbm.at[0, 0, pl.ds(0, ROWS), :], obuf, sem).wait()

    def compute_half(smbuf, half, obufa, obufb):
        @plsc.parallel_loop(0, HPIX, step=16, unroll=3)
        def _(base):
            q = jnp.right_shift(base, 4)
            rl = jnp.right_shift(q * 9363, 16)
            row = rl + half * 4
            col = base - rl * OW
            acc0 = jnp.zeros((16,), jnp.float32)
            acc1 = jnp.zeros((16,), jnp.float32)
            for k in range(K):
                xs = smbuf[pl.ds((2 * k) * HPIX + base, 16)]
                ys = smbuf[pl.ds((2 * k + 1) * HPIX + base, 16)]
                x0 = jnp.clip(xs.astype(jnp.int32), 0, W - 2)
                y0 = jnp.clip(ys.astype(jnp.int32), 0, H - 2)
                wx = xs - x0.astype(jnp.float32)
                wy = ys - y0.astype(jnp.float32)
                x1 = x0 + 1
                y1 = y0 + 1
                for pbuf, which in ((pba, 0), (pbb, 1)):
                    v00 = plsc.load_gather(pbuf, [y0, x0])
                    v01 = plsc.load_gather(pbuf, [y0, x1])
                    v10 = plsc.load_gather(pbuf, [y1, x0])
                    v11 = plsc.load_gather(pbuf, [y1, x1])
                    t0 = v00 + wx * (v01 - v00)
                    t1 = v10 + wx * (v11 - v10)
                    val = t0 + wy * (t1 - t0)
                    if which == 0:
                        acc0 = acc0 + val
                    else:
                        acc1 = acc1 + val
            obufa[row, pl.ds(col, 16)] = acc0 * 0.25
            obufb[row, pl.ds(col, 16)] = acc1 * 0.25

    def ppair_body(i, _):
        p0 = plane_base + i * 2
        ba, ca = bc_of(p0)
        bb, cb = bc_of(p0 + 1)
        pltpu.async_copy(x_hbm.at[ba, ca], pba, psem)
        pltpu.async_copy(x_hbm.at[bb, cb], pbb, psem)
        sm_start(0, sma, ssem_a)
        pltpu.make_async_copy(x_hbm.at[0, 0], pba, psem).wait()
        pltpu.make_async_copy(x_hbm.at[0, 0], pbb, psem).wait()

        def cpair_body(ci, _):
            h0 = ci * 4
            first = jnp.logical_and(i == 0, ci == 0)

            sm_start(h0 + 1, smb, ssem_b)
            sm_wait(sma, ssem_a)

            @pl.when(jnp.logical_not(first))
            def _():
                out_drain(oa0, osem_a0)
                out_drain(ob0, osem_b0)

            compute_half(sma, 0, oa0, ob0)
            sm_start(h0 + 2, sma, ssem_a)
            sm_wait(smb, ssem_b)
            compute_half(smb, 1, oa0, ob0)
            out_start(ba, ca, ci * 2, oa0, osem_a0)
            out_start(bb, cb, ci * 2, ob0, osem_b0)

            sm_start(h0 + 3, smb, ssem_b)
            sm_wait(sma, ssem_a)

            @pl.when(jnp.logical_not(first))
            def _():
                out_drain(oa1, osem_a1)
                out_drain(ob1, osem_b1)

            compute_half(sma, 0, oa1, ob1)

            @pl.when(ci < NCPAIR - 1)
            def _():
                sm_start(h0 + 4, sma, ssem_a)

            sm_wait(smb, ssem_b)
            compute_half(smb, 1, oa1, ob1)
            out_start(ba, ca, ci * 2 + 1, oa1, osem_a1)
            out_start(bb, cb, ci * 2 + 1, ob1, osem_b1)
            return 0

        lax.fori_loop(0, NCPAIR, cpair_body, 0)
        return 0

    lax.fori_loop(0, NPPAIR, ppair_body, 0)
    out_drain(oa0, osem_a0)
    out_drain(ob0, osem_b0)
    out_drain(oa1, osem_a1)
    out_drain(ob1, osem_b1)


@jax.jit
def _mapped_avg_pool_sc(x, sm_t):
    k = pl.kernel(
        _body,
        out_type=jax.ShapeDtypeStruct((B, C, OH, OW), jnp.float32),
        mesh=plsc.VectorSubcoreMesh(core_axis_name="c", subcore_axis_name="s"),
        scratch_types=[
            pltpu.VMEM((H, W), jnp.float32),
            pltpu.VMEM((H, W), jnp.float32),
            pltpu.VMEM((SMH,), jnp.float32),
            pltpu.VMEM((SMH,), jnp.float32),
            pltpu.VMEM((ROWS, OW), jnp.float32),
            pltpu.VMEM((ROWS, OW), jnp.float32),
            pltpu.VMEM((ROWS, OW), jnp.float32),
            pltpu.VMEM((ROWS, OW), jnp.float32),
            pltpu.SemaphoreType.DMA,
            pltpu.SemaphoreType.DMA,
            pltpu.SemaphoreType.DMA,
            pltpu.SemaphoreType.DMA,
            pltpu.SemaphoreType.DMA,
            pltpu.SemaphoreType.DMA,
            pltpu.SemaphoreType.DMA,
        ],
        compiler_params=pltpu.CompilerParams(needs_layout_passes=False),
    )
    return k(x, sm_t)


def kernel(x, sample_map):
    smf = sample_map.reshape(NCHUNK * 2, HPIX, 2 * K)
    sm_t = smf.transpose(0, 2, 1).reshape(NCHUNK * 2 * SMH)
    return _mapped_avg_pool_sc(x, sm_t)


# speedup vs baseline: 1.4111x; 1.0602x over previous
import jax
import jax.numpy as jnp
from jax import lax
from jax.experimental import pallas as pl
from jax.experimental.pallas import tpu as pltpu
from jax.experimental.pallas import tpu_sc as plsc

B, C, H, W = 2, 384, 224, 224
OH, OW, K = 112, 112, 4
BC = B * C
OP = OH * OW

NC, NS = 2, 16
NW = NC * NS
PLANES_PER_W = BC // NW
NPPAIR = PLANES_PER_W // 2
ROWS = 8
NCHUNK = OH // ROWS
CHUNK = ROWS * OW
HPIX = CHUNK // 2
SMH = 2 * K * HPIX
NCPAIR = NCHUNK // 2


def _body(x_hbm, sm_hbm, out_hbm,
          pba, pbb, sma, smb, oa0, oa1, ob0, ob1,
          psem, ssem_a, ssem_b, osem_a0, osem_a1, osem_b0, osem_b1):
    wid = lax.axis_index("s") * NC + lax.axis_index("c")
    plane_base = wid * PLANES_PER_W

    def bc_of(p):
        b = jnp.where(p >= C, 1, 0)
        return b, p - b * C

    def sm_start(h, buf, sem):
        pltpu.async_copy(sm_hbm.at[pl.ds(h * SMH, SMH)], buf, sem)

    def sm_wait(buf, sem):
        pltpu.make_async_copy(sm_hbm.at[pl.ds(0, SMH)], buf, sem).wait()

    def out_start(bb, cc, ch, obuf, sem):
        pltpu.async_copy(obuf, out_hbm.at[bb, cc, pl.ds(ch * ROWS, ROWS), :], sem)

    def out_drain(obuf, sem):
        pltpu.make_async_copy(out_hbm.at[0, 0, pl.ds(0, ROWS), :], obuf, sem).wait()

    def compute_half(smbuf, half, obufa, obufb):
        @plsc.parallel_loop(0, HPIX, step=16, unroll=3)
        def _(base):
            q = jnp.right_shift(base, 4)
            rl = jnp.right_shift(q * 9363, 16)
            row = rl + half * 4
            col = base - rl * OW
            acc0 = jnp.zeros((16,), jnp.float32)
            acc1 = jnp.zeros((16,), jnp.float32)
            for k in range(K):
                xs = smbuf[pl.ds((2 * k) * HPIX + base, 16)]
                ys = smbuf[pl.ds((2 * k + 1) * HPIX + base, 16)]
                x0 = xs.astype(jnp.int32)
                y0 = ys.astype(jnp.int32)
                wx = xs - x0.astype(jnp.float32)
                wy = ys - y0.astype(jnp.float32)
                x1 = x0 + 1
                y1 = y0 + 1
                for pbuf, which in ((pba, 0), (pbb, 1)):
                    v00 = plsc.load_gather(pbuf, [y0, x0])
                    v01 = plsc.load_gather(pbuf, [y0, x1])
                    v10 = plsc.load_gather(pbuf, [y1, x0])
                    v11 = plsc.load_gather(pbuf, [y1, x1])
                    t0 = v00 + wx * (v01 - v00)
                    t1 = v10 + wx * (v11 - v10)
                    val = t0 + wy * (t1 - t0)
                    if which == 0:
                        acc0 = acc0 + val
                    else:
                        acc1 = acc1 + val
            obufa[row, pl.ds(col, 16)] = acc0 * 0.25
            obufb[row, pl.ds(col, 16)] = acc1 * 0.25

    def ppair_body(i, _):
        p0 = plane_base + i * 2
        ba, ca = bc_of(p0)
        bb, cb = bc_of(p0 + 1)
        pltpu.async_copy(x_hbm.at[ba, ca], pba, psem)
        pltpu.async_copy(x_hbm.at[bb, cb], pbb, psem)
        sm_start(0, sma, ssem_a)
        pltpu.make_async_copy(x_hbm.at[0, 0], pba, psem).wait()
        pltpu.make_async_copy(x_hbm.at[0, 0], pbb, psem).wait()

        def cpair_body(ci, _):
            h0 = ci * 4
            first = jnp.logical_and(i == 0, ci == 0)

            sm_start(h0 + 1, smb, ssem_b)
            sm_wait(sma, ssem_a)

            @pl.when(jnp.logical_not(first))
            def _():
                out_drain(oa0, osem_a0)
                out_drain(ob0, osem_b0)

            compute_half(sma, 0, oa0, ob0)
            sm_start(h0 + 2, sma, ssem_a)
            sm_wait(smb, ssem_b)
            compute_half(smb, 1, oa0, ob0)
            out_start(ba, ca, ci * 2, oa0, osem_a0)
            out_start(bb, cb, ci * 2, ob0, osem_b0)

            sm_start(h0 + 3, smb, ssem_b)
            sm_wait(sma, ssem_a)

            @pl.when(jnp.logical_not(first))
            def _():
                out_drain(oa1, osem_a1)
                out_drain(ob1, osem_b1)

            compute_half(sma, 0, oa1, ob1)

            @pl.when(ci < NCPAIR - 1)
            def _():
                sm_start(h0 + 4, sma, ssem_a)

            sm_wait(smb, ssem_b)
            compute_half(smb, 1, oa1, ob1)
            out_start(ba, ca, ci * 2 + 1, oa1, osem_a1)
            out_start(bb, cb, ci * 2 + 1, ob1, osem_b1)
            return 0

        lax.fori_loop(0, NCPAIR, cpair_body, 0)
        return 0

    lax.fori_loop(0, NPPAIR, ppair_body, 0)
    out_drain(oa0, osem_a0)
    out_drain(ob0, osem_b0)
    out_drain(oa1, osem_a1)
    out_drain(ob1, osem_b1)


@jax.jit
def _mapped_avg_pool_sc(x, sm_t):
    k = pl.kernel(
        _body,
        out_type=jax.ShapeDtypeStruct((B, C, OH, OW), jnp.float32),
        mesh=plsc.VectorSubcoreMesh(core_axis_name="c", subcore_axis_name="s"),
        scratch_types=[
            pltpu.VMEM((H, W), jnp.float32),
            pltpu.VMEM((H, W), jnp.float32),
            pltpu.VMEM((SMH,), jnp.float32),
            pltpu.VMEM((SMH,), jnp.float32),
            pltpu.VMEM((ROWS, OW), jnp.float32),
            pltpu.VMEM((ROWS, OW), jnp.float32),
            pltpu.VMEM((ROWS, OW), jnp.float32),
            pltpu.VMEM((ROWS, OW), jnp.float32),
            pltpu.SemaphoreType.DMA,
            pltpu.SemaphoreType.DMA,
            pltpu.SemaphoreType.DMA,
            pltpu.SemaphoreType.DMA,
            pltpu.SemaphoreType.DMA,
            pltpu.SemaphoreType.DMA,
            pltpu.SemaphoreType.DMA,
        ],
        compiler_params=pltpu.CompilerParams(needs_layout_passes=False),
    )
    return k(x, sm_t)


def kernel(x, sample_map):
    smf = sample_map.reshape(NCHUNK * 2, HPIX, 2 * K)
    sm_t = smf.transpose(0, 2, 1).reshape(NCHUNK * 2 * SMH)
    return _mapped_avg_pool_sc(x, sm_t)


# speedup vs baseline: 1.4603x; 1.0349x over previous
import jax
import jax.numpy as jnp
from jax import lax
from jax.experimental import pallas as pl
from jax.experimental.pallas import tpu as pltpu
from jax.experimental.pallas import tpu_sc as plsc

B, C, H, W = 2, 384, 224, 224
OH, OW, K = 112, 112, 4
BC = B * C
OP = OH * OW

NC, NS = 2, 16
NW = NC * NS
PLANES_PER_W = BC // NW
NPPAIR = PLANES_PER_W // 2
ROWS = 8
NCHUNK = OH // ROWS
CHUNK = ROWS * OW
HPIX = CHUNK // 2
SMH = 2 * K * HPIX
NCPAIR = NCHUNK // 2


def _body(x_hbm, sm_hbm, out_hbm,
          pba, pbb, sma, smb, oa0, oa1, ob0, ob1,
          psem, ssem_a, ssem_b, osem_a0, osem_a1, osem_b0, osem_b1):
    wid = lax.axis_index("s") * NC + lax.axis_index("c")
    plane_base = wid * PLANES_PER_W

    def bc_of(p):
        b = jnp.where(p >= C, 1, 0)
        return b, p - b * C

    def sm_start(h, buf, sem):
        pltpu.async_copy(sm_hbm.at[pl.ds(h * SMH, SMH)], buf, sem)

    def sm_wait(buf, sem):
        pltpu.make_async_copy(sm_hbm.at[pl.ds(0, SMH)], buf, sem).wait()

    def out_start(bb, cc, ch, obuf, sem):
        pltpu.async_copy(obuf, out_hbm.at[bb, cc, pl.ds(ch * ROWS, ROWS), :], sem)

    def out_drain(obuf, sem):
        pltpu.make_async_copy(out_hbm.at[0, 0, pl.ds(0, ROWS), :], obuf, sem).wait()

    def compute_half(smbuf, half, obufa, obufb):
        @plsc.parallel_loop(0, HPIX, step=16, unroll=4)
        def _(base):
            q = jnp.right_shift(base, 4)
            rl = jnp.right_shift(q * 9363, 16)
            row = rl + half * 4
            col = base - rl * OW
            acc0 = jnp.zeros((16,), jnp.float32)
            acc1 = jnp.zeros((16,), jnp.float32)
            for k in range(K):
                xs = smbuf[pl.ds((2 * k) * HPIX + base, 16)]
                ys = smbuf[pl.ds((2 * k + 1) * HPIX + base, 16)]
                x0 = xs.astype(jnp.int32)
                y0 = ys.astype(jnp.int32)
                wx = xs - x0.astype(jnp.float32)
                wy = ys - y0.astype(jnp.float32)
                x1 = x0 + 1
                y1 = y0 + 1
                for pbuf, which in ((pba, 0), (pbb, 1)):
                    v00 = plsc.load_gather(pbuf, [y0, x0])
                    v01 = plsc.load_gather(pbuf, [y0, x1])
                    v10 = plsc.load_gather(pbuf, [y1, x0])
                    v11 = plsc.load_gather(pbuf, [y1, x1])
                    t0 = v00 + wx * (v01 - v00)
                    t1 = v10 + wx * (v11 - v10)
                    val = t0 + wy * (t1 - t0)
                    if which == 0:
                        acc0 = acc0 + val
                    else:
                        acc1 = acc1 + val
            obufa[row, pl.ds(col, 16)] = acc0 * 0.25
            obufb[row, pl.ds(col, 16)] = acc1 * 0.25

    def ppair_body(i, _):
        p0 = plane_base + i * 2
        ba, ca = bc_of(p0)
        bb, cb = bc_of(p0 + 1)
        pltpu.async_copy(x_hbm.at[ba, ca], pba, psem)
        pltpu.async_copy(x_hbm.at[bb, cb], pbb, psem)
        sm_start(0, sma, ssem_a)
        pltpu.make_async_copy(x_hbm.at[0, 0], pba, psem).wait()
        pltpu.make_async_copy(x_hbm.at[0, 0], pbb, psem).wait()

        def cpair_body(ci, _):
            h0 = ci * 4
            first = jnp.logical_and(i == 0, ci == 0)

            sm_start(h0 + 1, smb, ssem_b)
            sm_wait(sma, ssem_a)

            @pl.when(jnp.logical_not(first))
            def _():
                out_drain(oa0, osem_a0)
                out_drain(ob0, osem_b0)

            compute_half(sma, 0, oa0, ob0)
            sm_start(h0 + 2, sma, ssem_a)
            sm_wait(smb, ssem_b)
            compute_half(smb, 1, oa0, ob0)
            out_start(ba, ca, ci * 2, oa0, osem_a0)
            out_start(bb, cb, ci * 2, ob0, osem_b0)

            sm_start(h0 + 3, smb, ssem_b)
            sm_wait(sma, ssem_a)

            @pl.when(jnp.logical_not(first))
            def _():
                out_drain(oa1, osem_a1)
                out_drain(ob1, osem_b1)

            compute_half(sma, 0, oa1, ob1)

            @pl.when(ci < NCPAIR - 1)
            def _():
                sm_start(h0 + 4, sma, ssem_a)

            sm_wait(smb, ssem_b)
            compute_half(smb, 1, oa1, ob1)
            out_start(ba, ca, ci * 2 + 1, oa1, osem_a1)
            out_start(bb, cb, ci * 2 + 1, ob1, osem_b1)
            return 0

        lax.fori_loop(0, NCPAIR, cpair_body, 0)
        return 0

    lax.fori_loop(0, NPPAIR, ppair_body, 0)
    out_drain(oa0, osem_a0)
    out_drain(ob0, osem_b0)
    out_drain(oa1, osem_a1)
    out_drain(ob1, osem_b1)


@jax.jit
def _mapped_avg_pool_sc(x, sm_t):
    k = pl.kernel(
        _body,
        out_type=jax.ShapeDtypeStruct((B, C, OH, OW), jnp.float32),
        mesh=plsc.VectorSubcoreMesh(core_axis_name="c", subcore_axis_name="s"),
        scratch_types=[
            pltpu.VMEM((H, W), jnp.float32),
            pltpu.VMEM((H, W), jnp.float32),
            pltpu.VMEM((SMH,), jnp.float32),
            pltpu.VMEM((SMH,), jnp.float32),
            pltpu.VMEM((ROWS, OW), jnp.float32),
            pltpu.VMEM((ROWS, OW), jnp.float32),
            pltpu.VMEM((ROWS, OW), jnp.float32),
            pltpu.VMEM((ROWS, OW), jnp.float32),
            pltpu.SemaphoreType.DMA,
            pltpu.SemaphoreType.DMA,
            pltpu.SemaphoreType.DMA,
            pltpu.SemaphoreType.DMA,
            pltpu.SemaphoreType.DMA,
            pltpu.SemaphoreType.DMA,
            pltpu.SemaphoreType.DMA,
        ],
        compiler_params=pltpu.CompilerParams(needs_layout_passes=False),
    )
    return k(x, sm_t)


def kernel(x, sample_map):
    smf = sample_map.reshape(NCHUNK * 2, HPIX, 2 * K)
    sm_t = smf.transpose(0, 2, 1).reshape(NCHUNK * 2 * SMH)
    return _mapped_avg_pool_sc(x, sm_t)
